# baseline (speedup 1.0000x reference)
import functools

import jax
import jax.numpy as jnp
from jax import lax
from jax.experimental import pallas as pl
from jax.experimental.pallas import tpu as pltpu
from jax.experimental.pallas import tpu_sc as plsc


def _make_gather(V2, D, BATCH, T):
    W = 2 * D
    info = plsc.get_sparse_core_info()
    NC, NS = info.num_cores, info.num_subcores
    NW = NC * NS
    assert BATCH % (2 * NW) == 0 and T % 8 == 0
    rows_per_w = BATCH // NW
    idx_per_w = rows_per_w * T
    n16 = (T + 15) // 16

    mesh = plsc.VectorSubcoreMesh(core_axis_name="c", subcore_axis_name="s")

    @functools.partial(
        pl.kernel,
        out_type=jax.ShapeDtypeStruct((BATCH, T, D), jnp.float32),
        mesh=mesh,
        scratch_types=[
            pltpu.VMEM((idx_per_w + 16,), jnp.int32),
            pltpu.VMEM((n16 * 16,), jnp.int32),
            pltpu.VMEM((n16 * 16,), jnp.int32),
            pltpu.VMEM((T, W), jnp.float32),
            pltpu.VMEM((T, W), jnp.float32),
            pltpu.VMEM((T, D), jnp.float32),
            pltpu.VMEM((T, D), jnp.float32),
            pltpu.SemaphoreType.DMA,
            pltpu.SemaphoreType.DMA,
            pltpu.SemaphoreType.DMA,
            pltpu.SemaphoreType.DMA,
        ],
        compiler_params=pltpu.CompilerParams(use_tc_tiling_on_sc=True),
    )
    def gather_kernel(idx_hbm, table_hbm, out_hbm, idx_v, p0, p1, q0, q1,
                      o0, o1, sg0, sg1, sw0, sw1):
        pv = (p0, p1)
        pairs = (q0, q1)
        obuf = (o0, o1)
        sg = (sg0, sg1)
        sw = (sw0, sw1)
        wid = lax.axis_index("s") * NC + lax.axis_index("c")
        row0 = wid * rows_per_w

        pltpu.sync_copy(idx_hbm.at[pl.ds(wid * idx_per_w, idx_per_w)],
                        idx_v.at[pl.ds(0, idx_per_w)])

        def make_pairs(b, c):
            def one(i, carry):
                v = idx_v[pl.ds(c * T + i * 16, 16)]
                pv[b][pl.ds(i * 16, 16)] = lax.shift_right_logical(v, 1)
                return carry

            lax.fori_loop(0, n16, one, 0)

        def gather_desc(b):
            return pltpu.make_async_copy(
                table_hbm.at[pv[b].at[pl.ds(0, T)]], pairs[b], sg[b])

        def start_gather(b):
            pltpu.async_copy(
                table_hbm.at[pv[b].at[pl.ds(0, T)]], pairs[b], sg[b])

        for b in range(2):
            make_pairs(b, b)
            start_gather(b)

        def cycle(g, carry):
            for b in range(2):
                c = 2 * g + b
                gather_desc(b).wait()

                @pl.when(c >= 2)
                def _drain_write():
                    pltpu.make_async_copy(
                        obuf[b], out_hbm.at[row0 + c - 2], sw[b]).wait()

                def sel_rows(k0, nrows):
                    hv = (idx_v[pl.ds(c * T + k0, 16)] & 1) * D
                    for l in range(nrows):
                        h = hv[l]
                        for q in range(D // 16):
                            obuf[b][k0 + l, pl.ds(q * 16, 16)] = (
                                pairs[b][k0 + l, pl.ds(h + q * 16, 16)])

                def sel16(j, carry2):
                    sel_rows(j * 16, 16)
                    return carry2

                lax.fori_loop(0, (T // 16), sel16, 0)
                sel_rows((T // 16) * 16, T % 16)
                pltpu.async_copy(obuf[b], out_hbm.at[row0 + c], sw[b])

                @pl.when(c + 2 < rows_per_w)
                def _refill():
                    make_pairs(b, c + 2)
                    start_gather(b)

            return carry

        lax.fori_loop(0, rows_per_w // 2, cycle, 0)
        for b in range(2):
            c = rows_per_w - 2 + b
            pltpu.make_async_copy(obuf[b], out_hbm.at[row0 + c], sw[b]).wait()

    return gather_kernel


def kernel(inputs, embed_weight):
    batch, n_tokens = inputs.shape
    V, D = embed_weight.shape
    flat_idx = inputs.reshape(-1).astype(jnp.int32)
    table2 = embed_weight.reshape(V // 2, 2 * D)
    return _make_gather(V // 2, D, batch, n_tokens)(flat_idx, table2)


# speedup vs baseline: 1.0216x; 1.0216x over previous
import functools

import jax
import jax.numpy as jnp
from jax import lax
from jax.experimental import pallas as pl
from jax.experimental.pallas import tpu as pltpu
from jax.experimental.pallas import tpu_sc as plsc


def _make_gather(V, D, B):
    info = plsc.get_sparse_core_info()
    NC, NS = info.num_cores, info.num_subcores
    NW = NC * NS
    assert B % NW == 0
    b_per_w = B // NW
    C = 800
    assert b_per_w % (2 * C) == 0
    n_chunks = b_per_w // C
    n_pairs = n_chunks // 2
    assert b_per_w + 2 * C * D <= 131000

    mesh = plsc.VectorSubcoreMesh(core_axis_name="c", subcore_axis_name="s")

    @functools.partial(
        pl.kernel,
        out_type=jax.ShapeDtypeStruct((B, D), jnp.float32),
        mesh=mesh,
        scratch_types=[
            pltpu.VMEM((b_per_w,), jnp.int32),
            pltpu.VMEM((C, D), jnp.float32),
            pltpu.VMEM((C, D), jnp.float32),
            pltpu.SemaphoreType.DMA,
            pltpu.SemaphoreType.DMA,
            pltpu.SemaphoreType.DMA,
            pltpu.SemaphoreType.DMA,
        ],
        compiler_params=pltpu.CompilerParams(use_tc_tiling_on_sc=False),
    )
    def gather_kernel(idx_hbm, table_hbm, out_hbm, idx_v, rows0, rows1,
                      sg0, sg1, so0, so1):
        wid = lax.axis_index("s") * NC + lax.axis_index("c")
        base = wid * b_per_w
        pltpu.sync_copy(idx_hbm.at[pl.ds(base, b_per_w)], idx_v)

        def idx_slice(g):
            return idx_v.at[pl.ds(g * C, C)]

        def out_slice(g):
            return out_hbm.at[pl.ds(base + g * C, C)]

        pltpu.async_copy(table_hbm.at[idx_slice(0)], rows0, sg0)
        pltpu.async_copy(table_hbm.at[idx_slice(1)], rows1, sg1)

        def pair(i, carry):
            g = 2 * i
            pltpu.make_async_copy(table_hbm.at[idx_slice(g)], rows0, sg0).wait()
            pltpu.async_copy(rows0, out_slice(g), so0)
            pltpu.make_async_copy(table_hbm.at[idx_slice(g + 1)], rows1, sg1).wait()
            pltpu.async_copy(rows1, out_slice(g + 1), so1)

            @pl.when(i + 1 < n_pairs)
            def _refill():
                pltpu.make_async_copy(rows0, out_slice(g), so0).wait()
                pltpu.async_copy(table_hbm.at[idx_slice(g + 2)], rows0, sg0)
                pltpu.make_async_copy(rows1, out_slice(g + 1), so1).wait()
                pltpu.async_copy(table_hbm.at[idx_slice(g + 3)], rows1, sg1)

            return carry

        lax.fori_loop(0, n_pairs, pair, 0)
        pltpu.make_async_copy(rows0, out_slice(n_chunks - 2), so0).wait()
        pltpu.make_async_copy(rows1, out_slice(n_chunks - 1), so1).wait()

    return gather_kernel


def kernel(inputs, embed_weight):
    batch, n_tokens = inputs.shape
    V, D = embed_weight.shape
    flat_idx = inputs.reshape(-1).astype(jnp.int32)
    B = flat_idx.shape[0]
    out = _make_gather(V, D, B)(flat_idx, embed_weight)
    return out.reshape(batch, n_tokens, D)
